# baseline (speedup 1.0000x reference)
import jax
import jax.numpy as jnp
from jax.experimental import pallas as pl
from jax.experimental.pallas import tpu as pltpu

LANE = 128


def _fused_body(xt_ref, xst_ref, est_ref, wn_ref, we_ref, wr_ref, bn_ref,
                be_ref, br_ref, o_ref):
    fn, m2 = wn_ref.shape
    fe = we_ref.shape[0]
    d = xst_ref.shape[0] // fn
    inv_dp1 = 1.0 / (d + 1)
    f32 = jnp.float32

    wrn = wr_ref[:m2, :]
    wre = wr_ref[m2:, :]
    dims = (((0,), (1,)), ((), ()))
    wnf_t = jax.lax.dot_general(wrn, wn_ref[...], dims,
                                preferred_element_type=f32) * inv_dp1
    wef_t = jax.lax.dot_general(wre, we_ref[...], dims,
                                preferred_element_type=f32) * inv_dp1
    w2t = jnp.concatenate([wnf_t] * d, axis=1)
    w3t = jnp.repeat(wef_t, d, axis=1)
    bias = (jnp.dot(bn_ref[...], wrn, preferred_element_type=f32)
            + (d * inv_dp1) * jnp.dot(be_ref[...], wre,
                                      preferred_element_type=f32)
            + br_ref[...])

    acc = jnp.dot(wnf_t, xt_ref[...], preferred_element_type=f32)
    acc += jnp.dot(w2t, xst_ref[...], preferred_element_type=f32)
    acc += jnp.dot(w3t, est_ref[...], preferred_element_type=f32)
    o_ref[...] = acc + jnp.transpose(bias)


def _pick_lane_tile(n, *, max_tile=16384):
    best = None
    t = LANE
    while t <= min(max_tile, n // 2):
        if n % t == 0:
            best = t
        t += LANE
    return best if best is not None else n


def kernel(x, x_src, e_feat, wn_t, bn, we_t, be, wr_t, br):
    n, fn = x.shape
    _, d, fe = e_feat.shape
    m2 = wn_t.shape[1]
    r = wr_t.shape[1]

    xt = x.T
    xst = x_src.transpose(1, 2, 0).reshape(d * fn, n)
    est = e_feat.transpose(2, 1, 0).reshape(fe * d, n)

    tn = _pick_lane_tile(n)
    grid = n // tn

    k = fn + d * fn + fe * d
    flops = 2 * n * k * r + n * r
    bytes_accessed = 4 * (n * k + n * r + k * r + r)

    out_t = pl.pallas_call(
        _fused_body,
        out_shape=jax.ShapeDtypeStruct((r, n), jnp.float32),
        grid=(grid,),
        in_specs=[
            pl.BlockSpec((fn, tn), lambda i: (0, i)),
            pl.BlockSpec((d * fn, tn), lambda i: (0, i)),
            pl.BlockSpec((fe * d, tn), lambda i: (0, i)),
            pl.BlockSpec((fn, m2), lambda i: (0, 0)),
            pl.BlockSpec((fe, m2), lambda i: (0, 0)),
            pl.BlockSpec((2 * m2, r), lambda i: (0, 0)),
            pl.BlockSpec((1, m2), lambda i: (0, 0)),
            pl.BlockSpec((1, m2), lambda i: (0, 0)),
            pl.BlockSpec((1, r), lambda i: (0, 0)),
        ],
        out_specs=pl.BlockSpec((r, tn), lambda i: (0, i)),
        compiler_params=pltpu.CompilerParams(
            dimension_semantics=("parallel",),
            vmem_limit_bytes=32 * 1024 * 1024),
        cost_estimate=pl.CostEstimate(flops=flops, transcendentals=0,
                                      bytes_accessed=bytes_accessed),
    )(xt, xst, est, wn_t, we_t, wr_t, bn, be, br)
    return out_t.T


# speedup vs baseline: 1.0422x; 1.0422x over previous
import jax
import jax.numpy as jnp
from jax.experimental import pallas as pl
from jax.experimental.pallas import tpu as pltpu

LANE = 128


def _fused_body(xt_ref, xst_ref, est_ref, w1t_ref, w2t_ref, w3t_ref, bt_ref,
                o_ref):
    acc = jnp.dot(w1t_ref[...], xt_ref[...],
                  preferred_element_type=jnp.float32)
    acc += jnp.dot(w2t_ref[...], xst_ref[...],
                   preferred_element_type=jnp.float32)
    acc += jnp.dot(w3t_ref[...], est_ref[...],
                   preferred_element_type=jnp.float32)
    o_ref[...] = acc + bt_ref[...]


def _pick_lane_tile(n, *, max_tile=32768):
    best = None
    t = LANE
    while t <= min(max_tile, n // 2):
        if n % t == 0:
            best = t
        t += LANE
    return best if best is not None else n


def kernel(x, x_src, e_feat, wn_t, bn, we_t, be, wr_t, br):
    n, fn = x.shape
    _, d, fe = e_feat.shape
    m2 = wn_t.shape[1]
    r = wr_t.shape[1]
    hi = jax.lax.Precision.HIGHEST

    wrn, wre = wr_t[:m2], wr_t[m2:]
    inv_dp1 = 1.0 / (d + 1)
    wn_fold_t = jnp.dot(wrn.T, wn_t.T, precision=hi) * inv_dp1
    we_fold_t = jnp.dot(wre.T, we_t.T, precision=hi) * inv_dp1
    w1t = wn_fold_t
    w2t = jnp.tile(wn_fold_t, (1, d))
    w3t = jnp.repeat(we_fold_t, d, axis=1)
    bt = (jnp.dot(bn.reshape(1, m2), wrn, precision=hi)
          + (d * inv_dp1) * jnp.dot(be.reshape(1, m2), wre, precision=hi)
          + br.reshape(1, r)).reshape(r, 1)

    xt = x.T
    xst = x_src.transpose(1, 2, 0).reshape(d * fn, n)
    est = e_feat.transpose(2, 1, 0).reshape(fe * d, n)

    tn = _pick_lane_tile(n)
    grid = n // tn

    k = fn + d * fn + fe * d
    flops = 2 * n * k * r + n * r
    bytes_accessed = 4 * (n * k + n * r + k * r + r)

    out_t = pl.pallas_call(
        _fused_body,
        out_shape=jax.ShapeDtypeStruct((r, n), jnp.float32),
        grid=(grid,),
        in_specs=[
            pl.BlockSpec((fn, tn), lambda i: (0, i)),
            pl.BlockSpec((d * fn, tn), lambda i: (0, i)),
            pl.BlockSpec((fe * d, tn), lambda i: (0, i)),
            pl.BlockSpec((r, fn), lambda i: (0, 0)),
            pl.BlockSpec((r, d * fn), lambda i: (0, 0)),
            pl.BlockSpec((r, fe * d), lambda i: (0, 0)),
            pl.BlockSpec((r, 1), lambda i: (0, 0)),
        ],
        out_specs=pl.BlockSpec((r, tn), lambda i: (0, i)),
        compiler_params=pltpu.CompilerParams(
            dimension_semantics=("parallel",),
            vmem_limit_bytes=52 * 1024 * 1024),
        cost_estimate=pl.CostEstimate(flops=flops, transcendentals=0,
                                      bytes_accessed=bytes_accessed),
    )(xt, xst, est, w1t, w2t, w3t, bt)
    return out_t.T


# speedup vs baseline: 1.0439x; 1.0016x over previous
import jax
import jax.numpy as jnp
from jax.experimental import pallas as pl
from jax.experimental.pallas import tpu as pltpu

LANE = 128


def _fused_body(xt_ref, xst_ref, est_ref, w1t_ref, w2t_ref, w3t_ref, bt_ref,
                o_ref):
    acc = jnp.dot(w1t_ref[...], xt_ref[...],
                  preferred_element_type=jnp.float32)
    acc += jnp.dot(w2t_ref[...], xst_ref[...],
                   preferred_element_type=jnp.float32)
    acc += jnp.dot(w3t_ref[...], est_ref[...],
                   preferred_element_type=jnp.float32)
    o_ref[...] = acc + bt_ref[...]


def _pick_lane_tile(n, *, max_tile=16384):
    best = None
    t = LANE
    while t <= min(max_tile, n // 2):
        if n % t == 0:
            best = t
        t += LANE
    return best if best is not None else n


def kernel(x, x_src, e_feat, wn_t, bn, we_t, be, wr_t, br):
    n, fn = x.shape
    _, d, fe = e_feat.shape
    m2 = wn_t.shape[1]
    r = wr_t.shape[1]
    hi = jax.lax.Precision.HIGHEST

    wrn, wre = wr_t[:m2], wr_t[m2:]
    inv_dp1 = 1.0 / (d + 1)
    wn_fold_t = jnp.dot(wrn.T, wn_t.T, precision=hi) * inv_dp1
    we_fold_t = jnp.dot(wre.T, we_t.T, precision=hi) * inv_dp1
    w1t = wn_fold_t
    w2t = jnp.tile(wn_fold_t, (1, d))
    w3t = jnp.repeat(we_fold_t, d, axis=1)
    bvec = jnp.concatenate([bn.reshape(1, m2),
                            (d * inv_dp1) * be.reshape(1, m2)], axis=1)
    bt = (jnp.dot(bvec, wr_t, precision=hi) + br.reshape(1, r)).reshape(r, 1)

    xt = x.T
    xst = x_src.transpose(1, 2, 0).reshape(d * fn, n)
    est = e_feat.transpose(2, 1, 0).reshape(fe * d, n)

    tn = _pick_lane_tile(n)
    grid = n // tn

    k = fn + d * fn + fe * d
    flops = 2 * n * k * r + n * r
    bytes_accessed = 4 * (n * k + n * r + k * r + r)

    out_t = pl.pallas_call(
        _fused_body,
        out_shape=jax.ShapeDtypeStruct((r, n), jnp.float32),
        grid=(grid,),
        in_specs=[
            pl.BlockSpec((fn, tn), lambda i: (0, i)),
            pl.BlockSpec((d * fn, tn), lambda i: (0, i)),
            pl.BlockSpec((fe * d, tn), lambda i: (0, i)),
            pl.BlockSpec((r, fn), lambda i: (0, 0)),
            pl.BlockSpec((r, d * fn), lambda i: (0, 0)),
            pl.BlockSpec((r, fe * d), lambda i: (0, 0)),
            pl.BlockSpec((r, 1), lambda i: (0, 0)),
        ],
        out_specs=pl.BlockSpec((r, tn), lambda i: (0, i)),
        compiler_params=pltpu.CompilerParams(
            dimension_semantics=("parallel",),
            vmem_limit_bytes=32 * 1024 * 1024),
        cost_estimate=pl.CostEstimate(flops=flops, transcendentals=0,
                                      bytes_accessed=bytes_accessed),
    )(xt, xst, est, w1t, w2t, w3t, bt)
    return out_t.T
